# baseline (speedup 1.0000x reference)
import jax
import jax.numpy as jnp
from jax import lax
from jax.experimental import pallas as pl
from jax.experimental.pallas import tpu as pltpu
from jax.experimental.pallas import tpu_sc as plsc

_NC = 2
_NS = 16
_L = 16
_NW = _NC * _NS

_B = 16384
_D = 64
_K = 4
_ROW = _K * _D
_CHUNK_W = _B // _NW
_SUB = 128
_NSUB = _CHUNK_W // _SUB


def _sc_body(codes_hbm, pred_hbm, cent_hbm, out_hbm,
             codes2_v, idx2_v, rows2_v, out_v, sem0, sem1):
    c = lax.axis_index("c")
    s = lax.axis_index("s")
    wid = s * _NC + c
    wbase = wid * _CHUNK_W

    lanes = lax.iota(jnp.int32, _L)

    sems = [sem0, sem1]
    copies = [None, None]

    def start(sub):
        b = sub % 2
        pltpu.sync_copy(codes_hbm.at[:, pl.ds(wbase + sub * _SUB, _SUB)],
                        codes2_v.at[b])
        pltpu.sync_copy(pred_hbm.at[pl.ds(wbase + sub * _SUB, _SUB)],
                        idx2_v.at[b])
        copies[b] = pltpu.async_copy(cent_hbm.at[idx2_v.at[b]],
                                     rows2_v.at[b], sems[b])

    start(0)
    for sub in range(_NSUB):
        if sub + 1 < _NSUB:
            start(sub + 1)
        b = sub % 2
        copies[b].wait()

        @plsc.parallel_loop(0, _SUB, step=_L, carry=jnp.int32(0))
        def group(i0, carry, sub=sub, b=b):
            rowidx = [_L * j + lanes for j in range(4)]
            bv = jnp.zeros((_L,), jnp.float32)
            for u in range(_L):
                ivec = jnp.full((_L,), i0 + u, jnp.int32)
                xs = [plsc.load_gather(codes2_v.at[b], [rowidx[j], ivec])
                      for j in range(4)]
                best = None
                for k in range(_K):
                    t = None
                    for j in range(4):
                        e = jnp.abs(rows2_v[b, i0 + u,
                                            pl.ds(_D * k + _L * j, _L)]
                                    - xs[j])
                        t = e if t is None else t + e
                    sk = jnp.sum(t)
                    best = sk if best is None else jnp.minimum(best, sk)
                bv = jnp.where(lanes == u, best, bv)
            out_v[pl.ds(sub * _SUB + i0, _L)] = bv * (1.0 / _D)
            return carry

    pltpu.sync_copy(out_v, out_hbm.at[pl.ds(wbase, _CHUNK_W)])


@jax.jit
def _run(codes, pred, cent2d):
    mesh = plsc.VectorSubcoreMesh(core_axis_name="c", subcore_axis_name="s")
    f = pl.kernel(
        _sc_body,
        out_type=jax.ShapeDtypeStruct((_B,), jnp.float32),
        mesh=mesh,
        scratch_types=[
            pltpu.VMEM((2, _D, _SUB), jnp.float32),
            pltpu.VMEM((2, _SUB), jnp.int32),
            pltpu.VMEM((2, _SUB, _ROW), jnp.float32),
            pltpu.VMEM((_CHUNK_W,), jnp.float32),
            pltpu.SemaphoreType.DMA,
            pltpu.SemaphoreType.DMA,
        ],
        compiler_params=pltpu.CompilerParams(needs_layout_passes=False),
    )
    return f(codes, pred, cent2d)


def kernel(codes, pred_class, centroids):
    cent2d = centroids.reshape(centroids.shape[0], _ROW)
    return _run(jnp.transpose(codes), pred_class.astype(jnp.int32), cent2d)


# speedup vs baseline: 1.0615x; 1.0615x over previous
import jax
import jax.numpy as jnp
from jax import lax
from jax.experimental import pallas as pl
from jax.experimental.pallas import tpu as pltpu
from jax.experimental.pallas import tpu_sc as plsc

_NC = 2
_NS = 16
_L = 16
_NW = _NC * _NS

_B = 16384
_D = 64
_K = 4
_ROW = _K * _D
_CHUNK_W = _B // _NW
_SUB = 128
_NSUB = _CHUNK_W // _SUB


def _sc_body(codes_hbm, pred_hbm, cent_hbm, out_hbm,
             codes2_v, idx2_v, rows2_v, out_v, sem0, sem1):
    c = lax.axis_index("c")
    s = lax.axis_index("s")
    wid = s * _NC + c
    wbase = wid * _CHUNK_W

    lanes = lax.iota(jnp.int32, _L)

    sems = [sem0, sem1]
    copies = [None, None]

    def start(sub):
        b = sub % 2
        pltpu.sync_copy(codes_hbm.at[pl.ds(wbase + sub * _SUB, _SUB)],
                        codes2_v.at[b])
        pltpu.sync_copy(pred_hbm.at[pl.ds(wbase + sub * _SUB, _SUB)],
                        idx2_v.at[b])
        copies[b] = pltpu.async_copy(cent_hbm.at[idx2_v.at[b]],
                                     rows2_v.at[b], sems[b])

    start(0)
    for sub in range(_NSUB):
        if sub + 1 < _NSUB:
            start(sub + 1)
        b = sub % 2
        copies[b].wait()

        @plsc.parallel_loop(0, _SUB, step=_L, carry=jnp.int32(0))
        def group(i0, carry, sub=sub, b=b):
            bv = jnp.zeros((_L,), jnp.float32)
            for u in range(_L):
                xs = [codes2_v[b, i0 + u, pl.ds(_L * j, _L)]
                      for j in range(4)]
                best = None
                for k in range(_K):
                    t = None
                    for j in range(4):
                        e = jnp.abs(rows2_v[b, i0 + u,
                                            pl.ds(_D * k + _L * j, _L)]
                                    - xs[j])
                        t = e if t is None else t + e
                    sk = jnp.sum(t)
                    best = sk if best is None else jnp.minimum(best, sk)
                bv = jnp.where(lanes == u, best, bv)
            out_v[pl.ds(sub * _SUB + i0, _L)] = bv * (1.0 / _D)
            return carry

    pltpu.sync_copy(out_v, out_hbm.at[pl.ds(wbase, _CHUNK_W)])


@jax.jit
def _run(codes, pred, cent2d):
    mesh = plsc.VectorSubcoreMesh(core_axis_name="c", subcore_axis_name="s")
    f = pl.kernel(
        _sc_body,
        out_type=jax.ShapeDtypeStruct((_B,), jnp.float32),
        mesh=mesh,
        scratch_types=[
            pltpu.VMEM((2, _SUB, _D), jnp.float32),
            pltpu.VMEM((2, _SUB), jnp.int32),
            pltpu.VMEM((2, _SUB, _ROW), jnp.float32),
            pltpu.VMEM((_CHUNK_W,), jnp.float32),
            pltpu.SemaphoreType.DMA,
            pltpu.SemaphoreType.DMA,
        ],
        compiler_params=pltpu.CompilerParams(needs_layout_passes=False),
    )
    return f(codes, pred, cent2d)


def kernel(codes, pred_class, centroids):
    cent2d = centroids.reshape(centroids.shape[0], _ROW)
    return _run(codes, pred_class.astype(jnp.int32), cent2d)
